# baseline (speedup 1.0000x reference)
import functools

import jax
import jax.numpy as jnp
from jax import lax
from jax.experimental import pallas as pl
from jax.experimental.pallas import tpu as pltpu
from jax.experimental.pallas import tpu_sc as plsc

B = 16384
NC = 2
NS = 16
NW = NC * NS
PER_W = B // NW
W = 32
NCHUNK = PER_W // W

PI = 3.14159265358979
SC1 = -1.0 / 6.0
SC2 = 1.0 / 120.0
SC3 = -1.0 / 5040.0
MAGIC = 0x5F3759DF

_CPARAMS = dict(needs_layout_passes=False, use_tc_tiling_on_sc=True)


def _sinc_poly(x):
    xp = PI * x
    u = xp * xp
    return 1.0 + u * (SC1 + u * (SC2 + u * SC3))


def _inv_norm(svec):
    i = plsc.bitcast(svec, jnp.int32)
    y = plsc.bitcast(MAGIC - (i >> 1), jnp.float32)
    for _ in range(3):
        y = y * (1.5 - ((0.5 * svec) * y) * y)
    n = svec * y
    return 1.0 / jnp.maximum(n, 1e-12)


def _mesh():
    return plsc.VectorSubcoreMesh(
        core_axis_name="c", subcore_axis_name="s",
        num_cores=NC, num_subcores=NS)


def _make_period_kernel(scale, offset):
    scratch = [
        pltpu.VMEM((PER_W,), jnp.int32),
        pltpu.VMEM((PER_W,), jnp.int32),
        pltpu.VMEM((PER_W,), jnp.float32),
        pltpu.VMEM((6 * W, 128), jnp.float32),
        pltpu.VMEM((6 * W, 128), jnp.float32),
        pltpu.VMEM((W, 256), jnp.float32),
        pltpu.SemaphoreType.DMA,
        pltpu.SemaphoreType.DMA,
    ]

    @functools.partial(
        pl.kernel,
        out_type=jax.ShapeDtypeStruct((B, 256), jnp.float32),
        mesh=_mesh(),
        scratch_types=scratch,
        compiler_params=pltpu.CompilerParams(**_CPARAMS),
    )
    def pk(heads, tails, tvec, fp_h, fp_t, aa,
           out, heads_v, tails_v, tv_v, tb0, tb1, out_v, sem0, sem1):
        wid = lax.axis_index("s") * NC + lax.axis_index("c")
        base = pl.multiple_of(wid * PER_W, PER_W)

        pltpu.sync_copy(heads.at[pl.ds(base, PER_W)], heads_v)
        pltpu.sync_copy(tails.at[pl.ds(base, PER_W)], tails_v)
        pltpu.sync_copy(tvec.at[pl.ds(base, PER_W)], tv_v)

        def dma_list(c16, tb, sem):
            hs = heads_v.at[pl.ds(c16, W)]
            ts = tails_v.at[pl.ds(c16, W)]
            srcs = [(fp_h, hs, 0), (fp_h, ts, 1), (fp_t, ts, 2),
                    (fp_t, hs, 3), (aa, hs, 4), (aa, ts, 5)]
            return [pltpu.make_async_copy(
                        tab.at[ir], tb.at[pl.ds(s * W, W)], sem)
                    for tab, ir, s in srcs]

        def issue(c16, tb, sem):
            for cp in dma_list(c16, tb, sem):
                cp.start()

        def drain(c16, tb, sem):
            for cp in dma_list(c16, tb, sem):
                cp.wait()

        combo_src = [(0, 4, 0), (2, 5, 64), (1, 5, 0), (3, 4, 64)]

        def compute(c16, tb):
            def body(e, carry):
                li = c16 + e
                iv = jnp.full((16,), li, jnp.int32)
                traw = plsc.load_gather(tv_v, [iv])
                t = traw * scale + offset
                for combo in range(4):
                    fps, as_, ah = combo_src[combo]
                    for d in range(4):
                        f = tb[fps * W + e, pl.ds(d * 16, 16)]
                        p = tb[fps * W + e, pl.ds(64 + d * 16, 16)]
                        a = tb[as_ * W + e, pl.ds(ah + d * 16, 16)]
                        out_v[e, pl.ds(combo * 64 + d * 16, 16)] = (
                            a * _sinc_poly(f * t + p))
                return carry
            lax.fori_loop(0, W, body, 0)

        issue(0, tb0, sem0)

        def chunk_pair(cp, carry):
            c0 = pl.multiple_of(cp * (2 * W), W)
            c1 = pl.multiple_of(c0 + W, W)
            issue(c1, tb1, sem1)
            drain(c0, tb0, sem0)
            compute(c0, tb0)
            pltpu.sync_copy(out_v, out.at[pl.ds(base + c0, W)])

            @pl.when(cp < NCHUNK // 2 - 1)
            def _():
                issue(c0 + 2 * W, tb0, sem0)

            drain(c1, tb1, sem1)
            compute(c1, tb1)
            pltpu.sync_copy(out_v, out.at[pl.ds(base + c1, W)])
            return carry

        lax.fori_loop(0, NCHUNK // 2, chunk_pair, 0)

    return pk


def _make_final_kernel():
    scratch = [
        pltpu.VMEM((PER_W,), jnp.int32),
        pltpu.VMEM((PER_W,), jnp.int32),
        pltpu.VMEM((PER_W,), jnp.int32),
        pltpu.VMEM((2 * W, 128), jnp.float32),
        pltpu.VMEM((2 * W, 128), jnp.float32),
        pltpu.VMEM((2 * W, 128), jnp.float32),
        pltpu.VMEM((2 * W, 128), jnp.float32),
        pltpu.VMEM((3 * W, 256), jnp.float32),
        pltpu.VMEM((3 * W, 256), jnp.float32),
        pltpu.VMEM((PER_W,), jnp.float32),
        pltpu.SemaphoreType.DMA,
        pltpu.SemaphoreType.DMA,
    ]

    @functools.partial(
        pl.kernel,
        out_type=jax.ShapeDtypeStruct((B,), jnp.float32),
        mesh=_mesh(),
        scratch_types=scratch,
        compiler_params=pltpu.CompilerParams(**_CPARAMS),
    )
    def fk(heads, rels, tails, ee, rel_h_embs, rel_t_embs,
           acc_y, acc_m, acc_d,
           out, heads_v, tails_v, rels_v, eb0, eb1, rb0, rb1,
           ab0, ab1, out_v, sem0, sem1):
        wid = lax.axis_index("s") * NC + lax.axis_index("c")
        base = pl.multiple_of(wid * PER_W, PER_W)

        pltpu.sync_copy(heads.at[pl.ds(base, PER_W)], heads_v)
        pltpu.sync_copy(tails.at[pl.ds(base, PER_W)], tails_v)
        pltpu.sync_copy(rels.at[pl.ds(base, PER_W)], rels_v)

        def dma_list(c16, eb, rb, ab, sem):
            hs = heads_v.at[pl.ds(c16, W)]
            ts = tails_v.at[pl.ds(c16, W)]
            rs = rels_v.at[pl.ds(c16, W)]
            gbase = base + c16
            return [
                pltpu.make_async_copy(ee.at[hs], eb.at[pl.ds(0, W)], sem),
                pltpu.make_async_copy(ee.at[ts], eb.at[pl.ds(W, W)], sem),
                pltpu.make_async_copy(rel_h_embs.at[rs],
                                      rb.at[pl.ds(0, W)], sem),
                pltpu.make_async_copy(rel_t_embs.at[rs],
                                      rb.at[pl.ds(W, W)], sem),
                pltpu.make_async_copy(acc_y.at[pl.ds(gbase, W)],
                                      ab.at[pl.ds(0, W)], sem),
                pltpu.make_async_copy(acc_m.at[pl.ds(gbase, W)],
                                      ab.at[pl.ds(W, W)], sem),
                pltpu.make_async_copy(acc_d.at[pl.ds(gbase, W)],
                                      ab.at[pl.ds(2 * W, W)], sem),
            ]

        def issue(c16, eb, rb, ab, sem):
            for cp in dma_list(c16, eb, rb, ab, sem):
                cp.start()

        def drain(c16, eb, rb, ab, sem):
            for cp in dma_list(c16, eb, rb, ab, sem):
                cp.wait()

        lane0 = lax.iota(jnp.int32, 16) == 0
        ent_src = [(0, 0), (1, 64), (1, 0), (0, 64)]

        def compute(c16, eb, rb, ab):
            def body(e, carry):
                li = c16 + e
                iv = jnp.full((16,), li, jnp.int32)

                accs = []
                for combo in range(4):
                    acc = []
                    for d in range(4):
                        sl = pl.ds(combo * 64 + d * 16, 16)
                        acc.append(ab[e, sl] + ab[W + e, sl]
                                   + ab[2 * W + e, sl])
                    accs.append(acc)

                ents = []
                for k in range(4):
                    slot, half = ent_src[k]
                    ents.append([eb[slot * W + e, pl.ds(half + d * 16, 16)]
                                 for d in range(4)])

                invs = []
                for k in range(4):
                    sq = jnp.zeros((16,), jnp.float32)
                    for d in range(4):
                        sq = sq + ents[k][d] * ents[k][d]
                        sq = sq + accs[k][d] * accs[k][d]
                    s = jnp.sum(sq)
                    invs.append(_inv_norm(jnp.full((16,), s, jnp.float32)))

                sc = jnp.zeros((16,), jnp.float32)
                for d in range(4):
                    sl = pl.ds(d * 16, 16)
                    rh = rb[e, sl]
                    rt = rb[W + e, sl]
                    sc = sc + jnp.abs(ents[0][d] * invs[0] * rh
                                      - ents[1][d] * invs[1] * rt)
                    sc = sc + jnp.abs(ents[2][d] * invs[2] * rh
                                      - ents[3][d] * invs[3] * rt)
                for d in range(4):
                    sl = pl.ds(64 + d * 16, 16)
                    rh = rb[e, sl]
                    rt = rb[W + e, sl]
                    sc = sc + jnp.abs(accs[0][d] * invs[0] * rh
                                      - accs[1][d] * invs[1] * rt)
                    sc = sc + jnp.abs(accs[2][d] * invs[2] * rh
                                      - accs[3][d] * invs[3] * rt)
                res = 12.0 - jnp.sum(sc)
                plsc.store_scatter(out_v, [iv],
                                   jnp.full((16,), res, jnp.float32),
                                   mask=lane0)
                return carry
            lax.fori_loop(0, W, body, 0)

        issue(0, eb0, rb0, ab0, sem0)

        def chunk_pair(cp, carry):
            c0 = pl.multiple_of(cp * (2 * W), W)
            c1 = pl.multiple_of(c0 + W, W)
            issue(c1, eb1, rb1, ab1, sem1)
            drain(c0, eb0, rb0, ab0, sem0)
            compute(c0, eb0, rb0, ab0)

            @pl.when(cp < NCHUNK // 2 - 1)
            def _():
                issue(c0 + 2 * W, eb0, rb0, ab0, sem0)

            drain(c1, eb1, rb1, ab1, sem1)
            compute(c1, eb1, rb1, ab1)
            return carry

        lax.fori_loop(0, NCHUNK // 2, chunk_pair, 0)

        pltpu.sync_copy(out_v, out.at[pl.ds(base, PER_W)])

    return fk


_PK_Y = _make_period_kernel(1.0, -2010.0)
_PK_M = _make_period_kernel(1.0 / 6.0, -1.0)
_PK_D = _make_period_kernel(0.0625, -1.0)
_FK = _make_final_kernel()


def _pair(a, b):
    return jnp.concatenate([a, b], axis=1)


def kernel(heads, rels, tails, years, months, days,
           ent_embs_h, ent_embs_t, rel_h_embs, rel_t_embs,
           y_freq_h, y_freq_t, m_freq_h, m_freq_t, d_freq_h, d_freq_t,
           y_phi_h, y_phi_t, m_phi_h, m_phi_t, d_phi_h, d_phi_t,
           y_amps_h, y_amps_t, m_amps_h, m_amps_t, d_amps_h, d_amps_t):
    heads = heads.astype(jnp.int32)
    rels = rels.astype(jnp.int32)
    tails = tails.astype(jnp.int32)
    acc_y = _PK_Y(heads, tails, years,
                  _pair(y_freq_h, y_phi_h), _pair(y_freq_t, y_phi_t),
                  _pair(y_amps_h, y_amps_t))
    acc_m = _PK_M(heads, tails, months,
                  _pair(m_freq_h, m_phi_h), _pair(m_freq_t, m_phi_t),
                  _pair(m_amps_h, m_amps_t))
    acc_d = _PK_D(heads, tails, days,
                  _pair(d_freq_h, d_phi_h), _pair(d_freq_t, d_phi_t),
                  _pair(d_amps_h, d_amps_t))
    return _FK(heads, rels, tails, _pair(ent_embs_h, ent_embs_t),
               rel_h_embs, rel_t_embs, acc_y, acc_m, acc_d)


# speedup vs baseline: 1.2572x; 1.2572x over previous
import functools

import jax
import jax.numpy as jnp
from jax import lax
from jax.experimental import pallas as pl
from jax.experimental.pallas import tpu as pltpu
from jax.experimental.pallas import tpu_sc as plsc

B = 16384
NC = 2
NS = 16
NW = NC * NS
PER_W = B // NW
W = 16
NCHUNK = PER_W // W

PI = 3.14159265358979
SC1 = -1.0 / 6.0
SC2 = 1.0 / 120.0
SC3 = -1.0 / 5040.0
MAGIC = 0x5F3759DF

_CPARAMS = dict(needs_layout_passes=False, use_tc_tiling_on_sc=True)


def _sinc_poly(x):
    xp = PI * x
    u = xp * xp
    return 1.0 + u * (SC1 + u * (SC2 + u * SC3))


def _inv_norm(svec):
    i = plsc.bitcast(svec, jnp.int32)
    y = plsc.bitcast(MAGIC - (i >> 1), jnp.float32)
    for _ in range(3):
        y = y * (1.5 - ((0.5 * svec) * y) * y)
    n = svec * y
    return 1.0 / jnp.maximum(n, 1e-12)


def _mesh():
    return plsc.VectorSubcoreMesh(
        core_axis_name="c", subcore_axis_name="s",
        num_cores=NC, num_subcores=NS)


def _scalar_at(vref, li):
    v = plsc.load_gather(vref, [jnp.full((16,), li, jnp.int32)])
    return v[0]


def _make_period_kernel(scale, offset):
    scratch = [
        pltpu.VMEM((PER_W,), jnp.int32),
        pltpu.VMEM((PER_W,), jnp.int32),
        pltpu.VMEM((PER_W,), jnp.float32),
        pltpu.VMEM((12 * W, 64), jnp.float32),
        pltpu.VMEM((12 * W, 64), jnp.float32),
        pltpu.VMEM((W, 256), jnp.float32),
        pltpu.SemaphoreType.DMA,
        pltpu.SemaphoreType.DMA,
    ]

    @functools.partial(
        pl.kernel,
        out_type=jax.ShapeDtypeStruct((B, 256), jnp.float32),
        mesh=_mesh(),
        scratch_types=scratch,
        compiler_params=pltpu.CompilerParams(**_CPARAMS),
    )
    def pk(heads, tails, tvec, fh, ph_, ah, ft, pt, at_,
           out, heads_v, tails_v, tv_v, tb0, tb1, out_v, sem0, sem1):
        wid = lax.axis_index("s") * NC + lax.axis_index("c")
        base = pl.multiple_of(wid * PER_W, PER_W)

        pltpu.sync_copy(heads.at[pl.ds(base, PER_W)], heads_v)
        pltpu.sync_copy(tails.at[pl.ds(base, PER_W)], tails_v)
        pltpu.sync_copy(tvec.at[pl.ds(base, PER_W)], tv_v)

        head_tabs = [(fh, 0), (ph_, 1), (ah, 2),
                     (ft, 9), (pt, 10), (at_, 11)]
        tail_tabs = [(ft, 3), (pt, 4), (at_, 5),
                     (fh, 6), (ph_, 7), (ah, 8)]

        def issue(c16, tb, sem):
            def body(e, carry):
                li = c16 + e
                ih = _scalar_at(heads_v, li)
                it = _scalar_at(tails_v, li)
                for tab, t in head_tabs:
                    pltpu.async_copy(tab.at[ih], tb.at[t * W + e], sem)
                for tab, t in tail_tabs:
                    pltpu.async_copy(tab.at[it], tb.at[t * W + e], sem)
                return carry
            lax.fori_loop(0, W, body, 0)

        def drain(tb, sem):
            def wbody(i, carry):
                pltpu.make_async_copy(
                    fh.at[jnp.int32(0)], tb.at[jnp.int32(0)], sem).wait()
                return carry
            lax.fori_loop(0, 12 * W, wbody, 0)

        def compute(c16, tb):
            def body(e, carry):
                li = c16 + e
                iv = jnp.full((16,), li, jnp.int32)
                traw = plsc.load_gather(tv_v, [iv])
                t = traw * scale + offset
                for combo in range(4):
                    for d in range(4):
                        sl = pl.ds(d * 16, 16)
                        f = tb[(combo * 3 + 0) * W + e, sl]
                        p = tb[(combo * 3 + 1) * W + e, sl]
                        a = tb[(combo * 3 + 2) * W + e, sl]
                        out_v[e, pl.ds(combo * 64 + d * 16, 16)] = (
                            a * _sinc_poly(f * t + p))
                return carry
            lax.fori_loop(0, W, body, 0)

        issue(0, tb0, sem0)

        def chunk_pair(cp, carry):
            c0 = pl.multiple_of(cp * (2 * W), W)
            c1 = pl.multiple_of(c0 + W, W)
            issue(c1, tb1, sem1)
            drain(tb0, sem0)
            compute(c0, tb0)
            pltpu.sync_copy(out_v, out.at[pl.ds(base + c0, W)])

            @pl.when(cp < NCHUNK // 2 - 1)
            def _():
                issue(c0 + 2 * W, tb0, sem0)

            drain(tb1, sem1)
            compute(c1, tb1)
            pltpu.sync_copy(out_v, out.at[pl.ds(base + c1, W)])
            return carry

        lax.fori_loop(0, NCHUNK // 2, chunk_pair, 0)

    return pk


def _make_final_kernel():
    scratch = [
        pltpu.VMEM((PER_W,), jnp.int32),
        pltpu.VMEM((PER_W,), jnp.int32),
        pltpu.VMEM((PER_W,), jnp.int32),
        pltpu.VMEM((4 * W, 64), jnp.float32),
        pltpu.VMEM((4 * W, 64), jnp.float32),
        pltpu.VMEM((2 * W, 128), jnp.float32),
        pltpu.VMEM((2 * W, 128), jnp.float32),
        pltpu.VMEM((3 * W, 256), jnp.float32),
        pltpu.VMEM((3 * W, 256), jnp.float32),
        pltpu.VMEM((PER_W,), jnp.float32),
        pltpu.SemaphoreType.DMA,
        pltpu.SemaphoreType.DMA,
    ]

    @functools.partial(
        pl.kernel,
        out_type=jax.ShapeDtypeStruct((B,), jnp.float32),
        mesh=_mesh(),
        scratch_types=scratch,
        compiler_params=pltpu.CompilerParams(**_CPARAMS),
    )
    def fk(heads, rels, tails, ent_embs_h, ent_embs_t, rel_h_embs,
           rel_t_embs, acc_y, acc_m, acc_d,
           out, heads_v, tails_v, rels_v, eb0, eb1, rb0, rb1,
           ab0, ab1, out_v, sem0, sem1):
        wid = lax.axis_index("s") * NC + lax.axis_index("c")
        base = pl.multiple_of(wid * PER_W, PER_W)

        pltpu.sync_copy(heads.at[pl.ds(base, PER_W)], heads_v)
        pltpu.sync_copy(tails.at[pl.ds(base, PER_W)], tails_v)
        pltpu.sync_copy(rels.at[pl.ds(base, PER_W)], rels_v)

        ent_head = [(ent_embs_h, 0), (ent_embs_t, 3)]
        ent_tail = [(ent_embs_t, 1), (ent_embs_h, 2)]

        def issue(c16, eb, rb, ab, sem):
            def body(e, carry):
                li = c16 + e
                ih = _scalar_at(heads_v, li)
                it = _scalar_at(tails_v, li)
                for tab, k in ent_head:
                    pltpu.async_copy(tab.at[ih], eb.at[k * W + e], sem)
                for tab, k in ent_tail:
                    pltpu.async_copy(tab.at[it], eb.at[k * W + e], sem)
                return carry
            lax.fori_loop(0, W, body, 0)
            rs = rels_v.at[pl.ds(c16, W)]
            pltpu.async_copy(rel_h_embs.at[rs], rb.at[pl.ds(0, W)], sem)
            pltpu.async_copy(rel_t_embs.at[rs], rb.at[pl.ds(W, W)], sem)
            gbase = base + c16
            pltpu.async_copy(acc_y.at[pl.ds(gbase, W)], ab.at[pl.ds(0, W)], sem)
            pltpu.async_copy(acc_m.at[pl.ds(gbase, W)], ab.at[pl.ds(W, W)], sem)
            pltpu.async_copy(acc_d.at[pl.ds(gbase, W)],
                             ab.at[pl.ds(2 * W, W)], sem)

        def drain(eb, rb, ab, sem):
            def wbody(i, carry):
                pltpu.make_async_copy(
                    ent_embs_h.at[jnp.int32(0)], eb.at[jnp.int32(0)],
                    sem).wait()
                return carry
            lax.fori_loop(0, 4 * W, wbody, 0)
            rs0 = rels_v.at[pl.ds(0, W)]
            pltpu.make_async_copy(
                rel_h_embs.at[rs0], rb.at[pl.ds(0, W)], sem).wait()
            pltpu.make_async_copy(
                rel_t_embs.at[rs0], rb.at[pl.ds(W, W)], sem).wait()
            for j in range(3):
                pltpu.make_async_copy(
                    acc_y.at[pl.ds(0, W)], ab.at[pl.ds(j * W, W)], sem).wait()

        lane0 = lax.iota(jnp.int32, 16) == 0

        def compute(c16, eb, rb, ab):
            def body(e, carry):
                li = c16 + e
                iv = jnp.full((16,), li, jnp.int32)

                accs = []
                for combo in range(4):
                    acc = []
                    for d in range(4):
                        sl = pl.ds(combo * 64 + d * 16, 16)
                        acc.append(ab[e, sl] + ab[W + e, sl]
                                   + ab[2 * W + e, sl])
                    accs.append(acc)

                ents = []
                for k in range(4):
                    ents.append([eb[k * W + e, pl.ds(d * 16, 16)]
                                 for d in range(4)])

                invs = []
                for k in range(4):
                    sq = jnp.zeros((16,), jnp.float32)
                    for d in range(4):
                        sq = sq + ents[k][d] * ents[k][d]
                        sq = sq + accs[k][d] * accs[k][d]
                    s = jnp.sum(sq)
                    invs.append(_inv_norm(jnp.full((16,), s, jnp.float32)))

                sc = jnp.zeros((16,), jnp.float32)
                for d in range(4):
                    sl = pl.ds(d * 16, 16)
                    rh = rb[e, sl]
                    rt = rb[W + e, sl]
                    sc = sc + jnp.abs(ents[0][d] * invs[0] * rh
                                      - ents[1][d] * invs[1] * rt)
                    sc = sc + jnp.abs(ents[2][d] * invs[2] * rh
                                      - ents[3][d] * invs[3] * rt)
                for d in range(4):
                    sl = pl.ds(64 + d * 16, 16)
                    rh = rb[e, sl]
                    rt = rb[W + e, sl]
                    sc = sc + jnp.abs(accs[0][d] * invs[0] * rh
                                      - accs[1][d] * invs[1] * rt)
                    sc = sc + jnp.abs(accs[2][d] * invs[2] * rh
                                      - accs[3][d] * invs[3] * rt)
                res = 12.0 - jnp.sum(sc)
                plsc.store_scatter(out_v, [iv],
                                   jnp.full((16,), res, jnp.float32),
                                   mask=lane0)
                return carry
            lax.fori_loop(0, W, body, 0)

        issue(0, eb0, rb0, ab0, sem0)

        def chunk_pair(cp, carry):
            c0 = pl.multiple_of(cp * (2 * W), W)
            c1 = pl.multiple_of(c0 + W, W)
            issue(c1, eb1, rb1, ab1, sem1)
            drain(eb0, rb0, ab0, sem0)
            compute(c0, eb0, rb0, ab0)

            @pl.when(cp < NCHUNK // 2 - 1)
            def _():
                issue(c0 + 2 * W, eb0, rb0, ab0, sem0)

            drain(eb1, rb1, ab1, sem1)
            compute(c1, eb1, rb1, ab1)
            return carry

        lax.fori_loop(0, NCHUNK // 2, chunk_pair, 0)

        pltpu.sync_copy(out_v, out.at[pl.ds(base, PER_W)])

    return fk


_PK_Y = _make_period_kernel(1.0, -2010.0)
_PK_M = _make_period_kernel(1.0 / 6.0, -1.0)
_PK_D = _make_period_kernel(0.0625, -1.0)
_FK = _make_final_kernel()


def kernel(heads, rels, tails, years, months, days,
           ent_embs_h, ent_embs_t, rel_h_embs, rel_t_embs,
           y_freq_h, y_freq_t, m_freq_h, m_freq_t, d_freq_h, d_freq_t,
           y_phi_h, y_phi_t, m_phi_h, m_phi_t, d_phi_h, d_phi_t,
           y_amps_h, y_amps_t, m_amps_h, m_amps_t, d_amps_h, d_amps_t):
    heads = heads.astype(jnp.int32)
    rels = rels.astype(jnp.int32)
    tails = tails.astype(jnp.int32)
    acc_y = _PK_Y(heads, tails, years,
                  y_freq_h, y_phi_h, y_amps_h, y_freq_t, y_phi_t, y_amps_t)
    acc_m = _PK_M(heads, tails, months,
                  m_freq_h, m_phi_h, m_amps_h, m_freq_t, m_phi_t, m_amps_t)
    acc_d = _PK_D(heads, tails, days,
                  d_freq_h, d_phi_h, d_amps_h, d_freq_t, d_phi_t, d_amps_t)
    return _FK(heads, rels, tails, ent_embs_h, ent_embs_t,
               rel_h_embs, rel_t_embs, acc_y, acc_m, acc_d)


# speedup vs baseline: 1.4276x; 1.1355x over previous
import functools

import jax
import jax.numpy as jnp
from jax import lax
from jax.experimental import pallas as pl
from jax.experimental.pallas import tpu as pltpu
from jax.experimental.pallas import tpu_sc as plsc

B = 16384
NC = 2
NS = 16
NW = NC * NS
PER_W = B // NW
W = 16
NCHUNK = PER_W // W

PI = 3.14159265358979
SC1 = -1.0 / 6.0
SC2 = 1.0 / 120.0
SC3 = -1.0 / 5040.0
MAGIC = 0x5F3759DF

_CPARAMS = dict(needs_layout_passes=False, use_tc_tiling_on_sc=True)


def _sinc_poly(x):
    xp = PI * x
    u = xp * xp
    return 1.0 + u * (SC1 + u * (SC2 + u * SC3))


def _inv_norm(svec):
    i = plsc.bitcast(svec, jnp.int32)
    y = plsc.bitcast(MAGIC - (i >> 1), jnp.float32)
    for _ in range(3):
        y = y * (1.5 - ((0.5 * svec) * y) * y)
    n = svec * y
    return 1.0 / jnp.maximum(n, 1e-12)


def _mesh():
    return plsc.VectorSubcoreMesh(
        core_axis_name="c", subcore_axis_name="s",
        num_cores=NC, num_subcores=NS)


def _scalar_at(vref, li):
    v = plsc.load_gather(vref, [jnp.full((16,), li, jnp.int32)])
    return v[0]


def _make_pv_kernel(scale, offset):
    scratch = [
        pltpu.VMEM((PER_W,), jnp.int32),
        pltpu.VMEM((PER_W,), jnp.int32),
        pltpu.VMEM((PER_W,), jnp.float32),
        pltpu.VMEM((6 * W, 64), jnp.float32),
        pltpu.VMEM((6 * W, 64), jnp.float32),
        pltpu.VMEM((W, 128), jnp.float32),
        pltpu.SemaphoreType.DMA,
        pltpu.SemaphoreType.DMA,
    ]

    @functools.partial(
        pl.kernel,
        out_type=jax.ShapeDtypeStruct((B, 128), jnp.float32),
        mesh=_mesh(),
        scratch_types=scratch,
        compiler_params=pltpu.CompilerParams(**_CPARAMS),
    )
    def pk(heads, tails, tvec, fq, ph_, am,
           out, heads_v, tails_v, tv_v, tb0, tb1, out_v, sem0, sem1):
        wid = lax.axis_index("s") * NC + lax.axis_index("c")
        base = pl.multiple_of(wid * PER_W, PER_W)

        pltpu.sync_copy(heads.at[pl.ds(base, PER_W)], heads_v)
        pltpu.sync_copy(tails.at[pl.ds(base, PER_W)], tails_v)
        pltpu.sync_copy(tvec.at[pl.ds(base, PER_W)], tv_v)

        tabs = [fq, ph_, am]

        def issue(c16, tb, sem):
            def body(e, carry):
                li = c16 + e
                ih = _scalar_at(heads_v, li)
                it = _scalar_at(tails_v, li)
                for r, tab in enumerate(tabs):
                    pltpu.async_copy(tab.at[ih], tb.at[(r * 2) * W + e], sem)
                    pltpu.async_copy(tab.at[it],
                                     tb.at[(r * 2 + 1) * W + e], sem)
                return carry
            lax.fori_loop(0, W, body, 0)

        def drain(tb, sem):
            def wbody(i, carry):
                pltpu.make_async_copy(
                    fq.at[jnp.int32(0)], tb.at[jnp.int32(0)], sem).wait()
                return carry
            lax.fori_loop(0, 6 * W, wbody, 0)

        def compute(c16, tb):
            def body(e, carry):
                li = c16 + e
                iv = jnp.full((16,), li, jnp.int32)
                traw = plsc.load_gather(tv_v, [iv])
                t = traw * scale + offset
                for half in range(2):
                    for d in range(4):
                        sl = pl.ds(d * 16, 16)
                        f = tb[(0 * 2 + half) * W + e, sl]
                        p = tb[(1 * 2 + half) * W + e, sl]
                        a = tb[(2 * 2 + half) * W + e, sl]
                        out_v[e, pl.ds(half * 64 + d * 16, 16)] = (
                            a * _sinc_poly(f * t + p))
                return carry
            lax.fori_loop(0, W, body, 0)

        issue(0, tb0, sem0)

        def chunk_pair(cp, carry):
            c0 = pl.multiple_of(cp * (2 * W), W)
            c1 = pl.multiple_of(c0 + W, W)
            issue(c1, tb1, sem1)
            drain(tb0, sem0)
            compute(c0, tb0)
            pltpu.sync_copy(out_v, out.at[pl.ds(base + c0, W)])

            @pl.when(cp < NCHUNK // 2 - 1)
            def _():
                issue(c0 + 2 * W, tb0, sem0)

            drain(tb1, sem1)
            compute(c1, tb1)
            pltpu.sync_copy(out_v, out.at[pl.ds(base + c1, W)])
            return carry

        lax.fori_loop(0, NCHUNK // 2, chunk_pair, 0)

    return pk


def _make_final_kernel():
    scratch = [
        pltpu.VMEM((PER_W,), jnp.int32),
        pltpu.VMEM((PER_W,), jnp.int32),
        pltpu.VMEM((PER_W,), jnp.int32),
        pltpu.VMEM((4 * W, 64), jnp.float32),
        pltpu.VMEM((4 * W, 64), jnp.float32),
        pltpu.VMEM((2 * W, 128), jnp.float32),
        pltpu.VMEM((2 * W, 128), jnp.float32),
        pltpu.VMEM((6 * W, 128), jnp.float32),
        pltpu.VMEM((6 * W, 128), jnp.float32),
        pltpu.VMEM((PER_W,), jnp.float32),
        pltpu.SemaphoreType.DMA,
        pltpu.SemaphoreType.DMA,
    ]

    @functools.partial(
        pl.kernel,
        out_type=jax.ShapeDtypeStruct((B,), jnp.float32),
        mesh=_mesh(),
        scratch_types=scratch,
        compiler_params=pltpu.CompilerParams(**_CPARAMS),
    )
    def fk(heads, rels, tails, ent_embs_h, ent_embs_t, rel_h_embs,
           rel_t_embs, a_yh, a_mh, a_dh, a_yt, a_mt, a_dt,
           out, heads_v, tails_v, rels_v, eb0, eb1, rb0, rb1,
           ab0, ab1, out_v, sem0, sem1):
        wid = lax.axis_index("s") * NC + lax.axis_index("c")
        base = pl.multiple_of(wid * PER_W, PER_W)

        pltpu.sync_copy(heads.at[pl.ds(base, PER_W)], heads_v)
        pltpu.sync_copy(tails.at[pl.ds(base, PER_W)], tails_v)
        pltpu.sync_copy(rels.at[pl.ds(base, PER_W)], rels_v)

        ent_head = [(ent_embs_h, 0), (ent_embs_t, 3)]
        ent_tail = [(ent_embs_t, 1), (ent_embs_h, 2)]
        slabs = [a_yh, a_mh, a_dh, a_yt, a_mt, a_dt]

        def issue(c16, eb, rb, ab, sem):
            def body(e, carry):
                li = c16 + e
                ih = _scalar_at(heads_v, li)
                it = _scalar_at(tails_v, li)
                for tab, k in ent_head:
                    pltpu.async_copy(tab.at[ih], eb.at[k * W + e], sem)
                for tab, k in ent_tail:
                    pltpu.async_copy(tab.at[it], eb.at[k * W + e], sem)
                return carry
            lax.fori_loop(0, W, body, 0)
            rs = rels_v.at[pl.ds(c16, W)]
            pltpu.async_copy(rel_h_embs.at[rs], rb.at[pl.ds(0, W)], sem)
            pltpu.async_copy(rel_t_embs.at[rs], rb.at[pl.ds(W, W)], sem)
            gbase = base + c16
            for j, slab in enumerate(slabs):
                pltpu.async_copy(slab.at[pl.ds(gbase, W)],
                                 ab.at[pl.ds(j * W, W)], sem)

        def drain(eb, rb, ab, sem):
            def wbody(i, carry):
                pltpu.make_async_copy(
                    ent_embs_h.at[jnp.int32(0)], eb.at[jnp.int32(0)],
                    sem).wait()
                return carry
            lax.fori_loop(0, 4 * W, wbody, 0)
            rs0 = rels_v.at[pl.ds(0, W)]
            pltpu.make_async_copy(
                rel_h_embs.at[rs0], rb.at[pl.ds(0, W)], sem).wait()
            pltpu.make_async_copy(
                rel_t_embs.at[rs0], rb.at[pl.ds(W, W)], sem).wait()
            for j in range(6):
                pltpu.make_async_copy(
                    a_yh.at[pl.ds(0, W)], ab.at[pl.ds(j * W, W)], sem).wait()

        lane0 = lax.iota(jnp.int32, 16) == 0
        combo_src = [(0, 0), (3, 64), (0, 64), (3, 0)]

        def compute(c16, eb, rb, ab):
            def body(e, carry):
                li = c16 + e
                iv = jnp.full((16,), li, jnp.int32)

                accs = []
                for combo in range(4):
                    sb, col = combo_src[combo]
                    acc = []
                    for d in range(4):
                        sl = pl.ds(col + d * 16, 16)
                        acc.append(ab[(sb + 0) * W + e, sl]
                                   + ab[(sb + 1) * W + e, sl]
                                   + ab[(sb + 2) * W + e, sl])
                    accs.append(acc)

                ents = []
                for k in range(4):
                    ents.append([eb[k * W + e, pl.ds(d * 16, 16)]
                                 for d in range(4)])

                invs = []
                for k in range(4):
                    sq = jnp.zeros((16,), jnp.float32)
                    for d in range(4):
                        sq = sq + ents[k][d] * ents[k][d]
                        sq = sq + accs[k][d] * accs[k][d]
                    s = jnp.sum(sq)
                    invs.append(_inv_norm(jnp.full((16,), s, jnp.float32)))

                sc = jnp.zeros((16,), jnp.float32)
                for d in range(4):
                    sl = pl.ds(d * 16, 16)
                    rh = rb[e, sl]
                    rt = rb[W + e, sl]
                    sc = sc + jnp.abs(ents[0][d] * invs[0] * rh
                                      - ents[1][d] * invs[1] * rt)
                    sc = sc + jnp.abs(ents[2][d] * invs[2] * rh
                                      - ents[3][d] * invs[3] * rt)
                for d in range(4):
                    sl = pl.ds(64 + d * 16, 16)
                    rh = rb[e, sl]
                    rt = rb[W + e, sl]
                    sc = sc + jnp.abs(accs[0][d] * invs[0] * rh
                                      - accs[1][d] * invs[1] * rt)
                    sc = sc + jnp.abs(accs[2][d] * invs[2] * rh
                                      - accs[3][d] * invs[3] * rt)
                res = 12.0 - jnp.sum(sc)
                plsc.store_scatter(out_v, [iv],
                                   jnp.full((16,), res, jnp.float32),
                                   mask=lane0)
                return carry
            lax.fori_loop(0, W, body, 0)

        issue(0, eb0, rb0, ab0, sem0)

        def chunk_pair(cp, carry):
            c0 = pl.multiple_of(cp * (2 * W), W)
            c1 = pl.multiple_of(c0 + W, W)
            issue(c1, eb1, rb1, ab1, sem1)
            drain(eb0, rb0, ab0, sem0)
            compute(c0, eb0, rb0, ab0)

            @pl.when(cp < NCHUNK // 2 - 1)
            def _():
                issue(c0 + 2 * W, eb0, rb0, ab0, sem0)

            drain(eb1, rb1, ab1, sem1)
            compute(c1, eb1, rb1, ab1)
            return carry

        lax.fori_loop(0, NCHUNK // 2, chunk_pair, 0)

        pltpu.sync_copy(out_v, out.at[pl.ds(base, PER_W)])

    return fk


_PK_Y = _make_pv_kernel(1.0, -2010.0)
_PK_M = _make_pv_kernel(1.0 / 6.0, -1.0)
_PK_D = _make_pv_kernel(0.0625, -1.0)
_FK = _make_final_kernel()


def kernel(heads, rels, tails, years, months, days,
           ent_embs_h, ent_embs_t, rel_h_embs, rel_t_embs,
           y_freq_h, y_freq_t, m_freq_h, m_freq_t, d_freq_h, d_freq_t,
           y_phi_h, y_phi_t, m_phi_h, m_phi_t, d_phi_h, d_phi_t,
           y_amps_h, y_amps_t, m_amps_h, m_amps_t, d_amps_h, d_amps_t):
    heads = heads.astype(jnp.int32)
    rels = rels.astype(jnp.int32)
    tails = tails.astype(jnp.int32)
    a_yh = _PK_Y(heads, tails, years, y_freq_h, y_phi_h, y_amps_h)
    a_mh = _PK_M(heads, tails, months, m_freq_h, m_phi_h, m_amps_h)
    a_dh = _PK_D(heads, tails, days, d_freq_h, d_phi_h, d_amps_h)
    a_yt = _PK_Y(heads, tails, years, y_freq_t, y_phi_t, y_amps_t)
    a_mt = _PK_M(heads, tails, months, m_freq_t, m_phi_t, m_amps_t)
    a_dt = _PK_D(heads, tails, days, d_freq_t, d_phi_t, d_amps_t)
    return _FK(heads, rels, tails, ent_embs_h, ent_embs_t,
               rel_h_embs, rel_t_embs, a_yh, a_mh, a_dh, a_yt, a_mt, a_dt)
